# Initial kernel scaffold; baseline (speedup 1.0000x reference)
import jax
import jax.numpy as jnp
from jax.experimental import pallas as pl


def kernel(context, context_mask, query, reference_points, Wv, bv, Ws, bs, Wa, ba, Wo, bo):
    raise NotImplementedError("write your pallas kernel here")


# speedup vs baseline: 91.8439x; 91.8439x over previous
import functools

import jax
import jax.numpy as jnp
import numpy as np
from jax import lax
from jax.experimental import pallas as pl
from jax.experimental.pallas import tpu as pltpu
from jax.experimental.pallas import tpu_sc as plsc

B, NQ, DM, M, P, HH, WW = 4, 4096, 256, 8, 4, 64, 64
D = DM // M
MP = M * P
QB = 512
NPIX = HH * WW
QCHUNK = 1024
HALF = D // 2

_GSUM = np.kron(np.eye(M, dtype=np.float32),
                np.ones((P, P), dtype=np.float32))


def _axis_slots(loc):
    pix = loc * 64.0 - 0.5
    t0 = jnp.floor(pix)
    f = pix - t0
    w0 = 1.0 - f
    w1 = f
    v0 = (t0 >= 0.0) & (t0 <= 63.0)
    v1 = (t0 >= -1.0) & (t0 <= 62.0)
    base = jnp.clip(t0, 0.0, 62.0)
    ws0 = (jnp.where(v0 & (base == t0), w0, 0.0)
           + jnp.where(v1 & (base == t0 + 1.0), w1, 0.0))
    ws1 = (jnp.where(v0 & (base + 1.0 == t0), w0, 0.0)
           + jnp.where(v1 & (base == t0), w1, 0.0))
    return base.astype(jnp.int32), ws0, ws1


def _proj_body(q_ref, rp_ref, Wv_ref, bv_ref, Wsx_ref, bsx_ref, Wsy_ref,
               bsy_ref, Wa_ref, ba_ref, G_ref,
               val_ref, locx_ref, locy_ref, aw_ref, k0_ref,
               w0_ref, w1_ref, w2_ref, w3_ref):
    q = q_ref[0]
    val_ref[0] = jnp.dot(q, Wv_ref[...], preferred_element_type=jnp.float32) + bv_ref[...]

    offx = jnp.dot(q, Wsx_ref[...], preferred_element_type=jnp.float32) + bsx_ref[...]
    offy = jnp.dot(q, Wsy_ref[...], preferred_element_type=jnp.float32) + bsy_ref[...]
    rp = rp_ref[0]
    locx = rp[:, 0:1] + offx * (1.0 / WW)
    locy = rp[:, 1:2] + offy * (1.0 / HH)
    locx_ref[0] = locx
    locy_ref[0] = locy

    logits = jnp.dot(q, Wa_ref[...], preferred_element_type=jnp.float32) + ba_ref[...]
    logits = logits - jnp.max(logits, axis=-1, keepdims=True)
    e = jnp.exp(logits)
    denom = jnp.dot(e, G_ref[...], preferred_element_type=jnp.float32)
    aw = e / denom
    aw_ref[0] = aw

    bx, wsx0, wsx1 = _axis_slots(locx)
    by, wsy0, wsy1 = _axis_slots(locy)
    k0_ref[0] = by * WW + bx
    w0_ref[0] = aw * wsy0 * wsx0
    w1_ref[0] = aw * wsy0 * wsx1
    w2_ref[0] = aw * wsy1 * wsx0
    w3_ref[0] = aw * wsy1 * wsx1


def _proj(query, rp, Wv, bv, Wsx, bsx, Wsy, bsy, Wa, ba):
    grid = (B, NQ // QB)
    qspec = pl.BlockSpec((1, QB, DM), lambda b, i: (b, i, 0))
    small = pl.BlockSpec((1, QB, MP), lambda b, i: (b, i, 0))
    full2 = lambda shape: pl.BlockSpec(shape, lambda b, i: (0, 0))
    out_shapes = (
        jax.ShapeDtypeStruct((B, NQ, DM), jnp.float32),
        jax.ShapeDtypeStruct((B, NQ, MP), jnp.float32),
        jax.ShapeDtypeStruct((B, NQ, MP), jnp.float32),
        jax.ShapeDtypeStruct((B, NQ, MP), jnp.float32),
        jax.ShapeDtypeStruct((B, NQ, MP), jnp.int32),
        jax.ShapeDtypeStruct((B, NQ, MP), jnp.float32),
        jax.ShapeDtypeStruct((B, NQ, MP), jnp.float32),
        jax.ShapeDtypeStruct((B, NQ, MP), jnp.float32),
        jax.ShapeDtypeStruct((B, NQ, MP), jnp.float32),
    )
    return pl.pallas_call(
        _proj_body,
        grid=grid,
        in_specs=[
            qspec,
            pl.BlockSpec((1, QB, 2), lambda b, i: (b, i, 0)),
            full2((DM, DM)),
            full2((1, DM)),
            full2((DM, MP)),
            full2((1, MP)),
            full2((DM, MP)),
            full2((1, MP)),
            full2((DM, MP)),
            full2((1, MP)),
            full2((MP, MP)),
        ],
        out_specs=(pl.BlockSpec((1, QB, DM), lambda b, i: (b, i, 0)),
                   small, small, small, small, small, small, small, small),
        out_shape=out_shapes,
    )(query, rp, Wv, bv, Wsx, bsx, Wsy, bsy, Wa, ba, _GSUM)


def _outproj_body(x_ref, Wo_ref, bo_ref, o_ref):
    o_ref[0] = (jnp.dot(x_ref[0], Wo_ref[...], preferred_element_type=jnp.float32)
                + bo_ref[...])


def _outproj(x, Wo, bo):
    grid = (B, NQ // QB)
    return pl.pallas_call(
        _outproj_body,
        grid=grid,
        in_specs=[
            pl.BlockSpec((1, QB, DM), lambda b, i: (b, i, 0)),
            pl.BlockSpec((DM, DM), lambda b, i: (0, 0)),
            pl.BlockSpec((1, DM), lambda b, i: (0, 0)),
        ],
        out_specs=pl.BlockSpec((1, QB, DM), lambda b, i: (b, i, 0)),
        out_shape=jax.ShapeDtypeStruct((B, NQ, DM), jnp.float32),
    )(x, Wo, bo)


def _sc_body(val_t, k0_t, w_t, out_hbm, table_v, idx_v, wv_v, out_v):
    wid = lax.axis_index("s") * 2 + lax.axis_index("c")
    b = wid // M
    m = wid % M

    for half in range(2):
        pltpu.sync_copy(val_t.at[b, m, pl.ds(half * HALF * NPIX, HALF * NPIX)],
                        table_v)

        def chunk_body(chunk, carry):
            cb = chunk * QCHUNK
            pltpu.sync_copy(k0_t.at[b, m, :, pl.ds(cb, QCHUNK)], idx_v)
            pltpu.sync_copy(w_t.at[b, m, :, :, pl.ds(cb, QCHUNK)], wv_v)

            def g_body(g, carry2):
                gb = g * 16
                acc = [jnp.zeros((16,), jnp.float32) for _ in range(HALF)]
                for p in range(P):
                    kv = idx_v[p, pl.ds(gb, 16)]
                    for s, off in enumerate((0, 1, WW, WW + 1)):
                        idx = kv + off
                        wv = wv_v[p, s, pl.ds(gb, 16)]
                        for d in range(HALF):
                            gathered = plsc.load_gather(table_v, [idx + d * NPIX])
                            acc[d] = acc[d] + wv * gathered
                for d in range(HALF):
                    out_v[d, pl.ds(gb, 16)] = acc[d]
                return carry2

            lax.fori_loop(0, QCHUNK // 16, g_body, 0)
            pltpu.sync_copy(
                out_v, out_hbm.at[b, m, pl.ds(half * HALF, HALF), pl.ds(cb, QCHUNK)])
            return carry

        lax.fori_loop(0, NQ // QCHUNK, chunk_body, 0)


def _sc_gather(val_t, k0_t, w_t):
    mesh = plsc.VectorSubcoreMesh(core_axis_name="c", subcore_axis_name="s")
    fn = functools.partial(
        pl.kernel,
        mesh=mesh,
        compiler_params=pltpu.CompilerParams(needs_layout_passes=False),
        out_type=jax.ShapeDtypeStruct((B, M, D, NQ), jnp.float32),
        scratch_types=[
            pltpu.VMEM((HALF * NPIX,), jnp.float32),
            pltpu.VMEM((P, QCHUNK), jnp.int32),
            pltpu.VMEM((P, 4, QCHUNK), jnp.float32),
            pltpu.VMEM((HALF, QCHUNK), jnp.float32),
        ],
    )(_sc_body)
    return fn(val_t, k0_t, w_t)


def kernel(context, context_mask, query, reference_points, Wv, bv, Ws, bs,
           Wa, ba, Wo, bo):
    rp = reference_points.reshape(B, NQ, 2)
    Wsx = Ws[:, 0::2]
    Wsy = Ws[:, 1::2]
    bsx = bs[0::2].reshape(1, MP)
    bsy = bs[1::2].reshape(1, MP)

    (value, locx, locy, aw, k0, w0, w1, w2, w3) = _proj(
        query, rp, Wv, bv.reshape(1, DM), Wsx, bsx, Wsy, bsy,
        Wa, ba.reshape(1, MP))

    val_t = (value.reshape(B, NQ, M, D).transpose(0, 2, 3, 1)
             .reshape(B, M, D * NQ))
    k0_t = k0.reshape(B, NQ, M, P).transpose(0, 2, 3, 1)
    w_t = (jnp.stack([w0, w1, w2, w3], axis=-1)
           .reshape(B, NQ, M, P, 4).transpose(0, 2, 3, 4, 1))

    outg_t = _sc_gather(val_t, k0_t, w_t)
    outg = outg_t.transpose(0, 3, 1, 2).reshape(B, NQ, DM)

    final = _outproj(outg, Wo, bo.reshape(1, DM))

    sampling_locations = (jnp.stack([locx, locy], axis=-1)
                          .reshape(B, NQ, M, 1, P, 2))
    attention_weights = aw.reshape(B, NQ, M, 1, P)
    return (final, sampling_locations, attention_weights)


# speedup vs baseline: 115.4068x; 1.2566x over previous
import functools

import jax
import jax.numpy as jnp
import numpy as np
from jax import lax
from jax.experimental import pallas as pl
from jax.experimental.pallas import tpu as pltpu
from jax.experimental.pallas import tpu_sc as plsc

B, NQ, DM, M, P, HH, WW = 4, 4096, 256, 8, 4, 64, 64
D = DM // M
MP = M * P
QB = 512
NPIX = HH * WW
QCHUNK = 1024
HALF = D // 2

_GSUM = np.kron(np.eye(M, dtype=np.float32),
                np.ones((P, P), dtype=np.float32))

_TDIMS = (((0,), (1,)), ((), ()))
_CDIMS = (((0,), (0,)), ((), ()))


def _axis_slots(loc):
    pix = loc * 64.0 - 0.5
    t0 = jnp.floor(pix)
    f = pix - t0
    w0 = 1.0 - f
    w1 = f
    v0 = (t0 >= 0.0) & (t0 <= 63.0)
    v1 = (t0 >= -1.0) & (t0 <= 62.0)
    base = jnp.clip(t0, 0.0, 62.0)
    ws0 = (jnp.where(v0 & (base == t0), w0, 0.0)
           + jnp.where(v1 & (base == t0 + 1.0), w1, 0.0))
    ws1 = (jnp.where(v0 & (base + 1.0 == t0), w0, 0.0)
           + jnp.where(v1 & (base == t0), w1, 0.0))
    return base.astype(jnp.int32), ws0, ws1


def _proj_body(q_ref, rp_ref, Wv_ref, bv_ref, Wsx_ref, bsx_ref, Wsy_ref,
               bsy_ref, Wa_ref, ba_ref, G_ref,
               val_ref, k0_ref, w0_ref, w1_ref, w2_ref, w3_ref,
               locx_ref, locy_ref, aw_ref):
    q = q_ref[0]
    val_ref[0] = (lax.dot_general(Wv_ref[...], q, _TDIMS,
                                  preferred_element_type=jnp.float32)
                  + bv_ref[...])

    offx = (lax.dot_general(Wsx_ref[...], q, _TDIMS,
                            preferred_element_type=jnp.float32) + bsx_ref[...])
    offy = (lax.dot_general(Wsy_ref[...], q, _TDIMS,
                            preferred_element_type=jnp.float32) + bsy_ref[...])
    rp = rp_ref[0]
    locx = rp[0:1, :] + offx * (1.0 / WW)
    locy = rp[1:2, :] + offy * (1.0 / HH)
    locx_ref[0] = locx.T
    locy_ref[0] = locy.T

    logits = (lax.dot_general(Wa_ref[...], q, _TDIMS,
                              preferred_element_type=jnp.float32) + ba_ref[...])
    logits = logits - jnp.max(logits, axis=0, keepdims=True)
    e = jnp.exp(logits)
    denom = jnp.dot(G_ref[...], e, preferred_element_type=jnp.float32)
    aw = e / denom
    aw_ref[0] = aw.T

    bx, wsx0, wsx1 = _axis_slots(locx)
    by, wsy0, wsy1 = _axis_slots(locy)
    k0_ref[0] = by * WW + bx
    w0_ref[0] = aw * wsy0 * wsx0
    w1_ref[0] = aw * wsy0 * wsx1
    w2_ref[0] = aw * wsy1 * wsx0
    w3_ref[0] = aw * wsy1 * wsx1


def _proj(query, rp_T, Wv, bv, Wsx, bsx, Wsy, bsy, Wa, ba):
    grid = (B, NQ // QB)
    tspec = pl.BlockSpec((1, MP, QB), lambda b, i: (b, 0, i))
    uspec = pl.BlockSpec((1, QB, MP), lambda b, i: (b, i, 0))
    full2 = lambda shape: pl.BlockSpec(shape, lambda b, i: (0, 0))
    out_shapes = (
        jax.ShapeDtypeStruct((B, DM, NQ), jnp.float32),
        jax.ShapeDtypeStruct((B, MP, NQ), jnp.int32),
        jax.ShapeDtypeStruct((B, MP, NQ), jnp.float32),
        jax.ShapeDtypeStruct((B, MP, NQ), jnp.float32),
        jax.ShapeDtypeStruct((B, MP, NQ), jnp.float32),
        jax.ShapeDtypeStruct((B, MP, NQ), jnp.float32),
        jax.ShapeDtypeStruct((B, NQ, MP), jnp.float32),
        jax.ShapeDtypeStruct((B, NQ, MP), jnp.float32),
        jax.ShapeDtypeStruct((B, NQ, MP), jnp.float32),
    )
    return pl.pallas_call(
        _proj_body,
        grid=grid,
        in_specs=[
            pl.BlockSpec((1, QB, DM), lambda b, i: (b, i, 0)),
            pl.BlockSpec((1, 2, QB), lambda b, i: (b, 0, i)),
            full2((DM, DM)),
            full2((DM, 1)),
            full2((DM, MP)),
            full2((MP, 1)),
            full2((DM, MP)),
            full2((MP, 1)),
            full2((DM, MP)),
            full2((MP, 1)),
            full2((MP, MP)),
        ],
        out_specs=(pl.BlockSpec((1, DM, QB), lambda b, i: (b, 0, i)),
                   tspec, tspec, tspec, tspec, tspec,
                   uspec, uspec, uspec),
        out_shape=out_shapes,
    )(query, rp_T, Wv, bv, Wsx, bsx, Wsy, bsy, Wa, ba, _GSUM)


def _outproj_body(x_ref, Wo_ref, bo_ref, o_ref):
    o_ref[0] = (lax.dot_general(x_ref[0], Wo_ref[...], _CDIMS,
                                preferred_element_type=jnp.float32)
                + bo_ref[...])


def _outproj(x_T, Wo, bo):
    grid = (B, NQ // QB)
    return pl.pallas_call(
        _outproj_body,
        grid=grid,
        in_specs=[
            pl.BlockSpec((1, DM, QB), lambda b, i: (b, 0, i)),
            pl.BlockSpec((DM, DM), lambda b, i: (0, 0)),
            pl.BlockSpec((1, DM), lambda b, i: (0, 0)),
        ],
        out_specs=pl.BlockSpec((1, QB, DM), lambda b, i: (b, i, 0)),
        out_shape=jax.ShapeDtypeStruct((B, NQ, DM), jnp.float32),
    )(x_T, Wo, bo)


def _sc_body(val_t, k0_t, w0_t, w1_t, w2_t, w3_t, out_hbm,
             table_v, idx_v, wv_v, out_v):
    wid = lax.axis_index("s") * 2 + lax.axis_index("c")
    b = wid // M
    m = wid % M

    for half in range(2):
        pltpu.sync_copy(
            val_t.at[b, pl.ds((m * D + half * HALF) * NQ, HALF * NPIX)],
            table_v)

        def chunk_body(chunk, carry):
            cb = chunk * QCHUNK
            pltpu.sync_copy(k0_t.at[b, pl.ds(m * P, P), pl.ds(cb, QCHUNK)],
                            idx_v)
            for s, w_t in enumerate((w0_t, w1_t, w2_t, w3_t)):
                pltpu.sync_copy(w_t.at[b, pl.ds(m * P, P), pl.ds(cb, QCHUNK)],
                                wv_v.at[s])

            def g_body(g, carry2):
                gb = g * 16
                acc = [jnp.zeros((16,), jnp.float32) for _ in range(HALF)]
                for p in range(P):
                    kv = idx_v[p, pl.ds(gb, 16)]
                    for s, off in enumerate((0, 1, WW, WW + 1)):
                        idx = kv + off
                        wv = wv_v[s, p, pl.ds(gb, 16)]
                        for d in range(HALF):
                            gathered = plsc.load_gather(
                                table_v, [idx + d * NPIX])
                            acc[d] = acc[d] + wv * gathered
                for d in range(HALF):
                    out_v[d, pl.ds(gb, 16)] = acc[d]
                return carry2

            lax.fori_loop(0, QCHUNK // 16, g_body, 0)
            pltpu.sync_copy(
                out_v,
                out_hbm.at[b, pl.ds(m * D + half * HALF, HALF),
                           pl.ds(cb, QCHUNK)])
            return carry

        lax.fori_loop(0, NQ // QCHUNK, chunk_body, 0)


def _sc_gather(val_t, k0_t, w0_t, w1_t, w2_t, w3_t):
    mesh = plsc.VectorSubcoreMesh(core_axis_name="c", subcore_axis_name="s")
    fn = functools.partial(
        pl.kernel,
        mesh=mesh,
        compiler_params=pltpu.CompilerParams(needs_layout_passes=False),
        out_type=jax.ShapeDtypeStruct((B, DM, NQ), jnp.float32),
        scratch_types=[
            pltpu.VMEM((HALF * NPIX,), jnp.float32),
            pltpu.VMEM((P, QCHUNK), jnp.int32),
            pltpu.VMEM((4, P, QCHUNK), jnp.float32),
            pltpu.VMEM((HALF, QCHUNK), jnp.float32),
        ],
    )(_sc_body)
    return fn(val_t.reshape(B, DM * NQ), k0_t, w0_t, w1_t, w2_t, w3_t)


def kernel(context, context_mask, query, reference_points, Wv, bv, Ws, bs,
           Wa, ba, Wo, bo):
    rp_T = reference_points.reshape(B, NQ, 2).transpose(0, 2, 1)
    Wsx = Ws[:, 0::2]
    Wsy = Ws[:, 1::2]
    bsx = bs[0::2].reshape(MP, 1)
    bsy = bs[1::2].reshape(MP, 1)

    (val_T, k0_T, w0_T, w1_T, w2_T, w3_T, locx, locy, aw) = _proj(
        query, rp_T, Wv, bv.reshape(DM, 1), Wsx, bsx, Wsy, bsy,
        Wa, ba.reshape(MP, 1))

    outg_T = _sc_gather(val_T, k0_T, w0_T, w1_T, w2_T, w3_T)

    final = _outproj(outg_T, Wo, bo.reshape(1, DM))

    sampling_locations = (jnp.stack([locx, locy], axis=-1)
                          .reshape(B, NQ, M, 1, P, 2))
    attention_weights = aw.reshape(B, NQ, M, 1, P)
    return (final, sampling_locations, attention_weights)
